# baseline (speedup 1.0000x reference)
import jax
import jax.numpy as jnp
from jax.experimental import pallas as pl
from jax.experimental.pallas import tpu as pltpu


def _add_kernel(patch_ref, pos_ref, out_ref):
    out_ref[...] = patch_ref[...] + pos_ref[...]


def kernel(patch, pos_table):
    B, N, D = patch.shape
    CN = 192
    return pl.pallas_call(
        _add_kernel,
        grid=(B, N // CN),
        in_specs=[
            pl.BlockSpec((1, CN, D), lambda b, n: (b, n, 0)),
            pl.BlockSpec((CN, D), lambda b, n: (n, 0)),
        ],
        out_specs=pl.BlockSpec((1, CN, D), lambda b, n: (b, n, 0)),
        out_shape=jax.ShapeDtypeStruct((B, N, D), patch.dtype),
        compiler_params=pltpu.CompilerParams(
            dimension_semantics=("parallel", "parallel"),
        ),
    )(patch, pos_table)


# speedup vs baseline: 2.4049x; 2.4049x over previous
import jax
import jax.numpy as jnp
from jax.experimental import pallas as pl
from jax.experimental.pallas import tpu as pltpu


def _add_kernel(patch_ref, pos_ref, out_ref):
    out_ref[...] = patch_ref[...] + pos_ref[...]


def kernel(patch, pos_table):
    B, N, D = patch.shape
    CB = 4
    return pl.pallas_call(
        _add_kernel,
        grid=(B // CB,),
        in_specs=[
            pl.BlockSpec((CB, N, D), lambda b: (b, 0, 0)),
            pl.BlockSpec((N, D), lambda b: (0, 0)),
        ],
        out_specs=pl.BlockSpec((CB, N, D), lambda b: (b, 0, 0)),
        out_shape=jax.ShapeDtypeStruct((B, N, D), patch.dtype),
        compiler_params=pltpu.CompilerParams(
            dimension_semantics=("parallel",),
        ),
    )(patch, pos_table)


# speedup vs baseline: 2.4372x; 1.0134x over previous
import jax
import jax.numpy as jnp
from jax.experimental import pallas as pl
from jax.experimental.pallas import tpu as pltpu


def _add_kernel(patch_ref, pos_ref, out_ref):
    out_ref[...] = patch_ref[...] + pos_ref[...]


def kernel(patch, pos_table):
    B, N, D = patch.shape
    CB = 8
    return pl.pallas_call(
        _add_kernel,
        grid=(B // CB,),
        in_specs=[
            pl.BlockSpec((CB, N, D), lambda b: (b, 0, 0)),
            pl.BlockSpec((N, D), lambda b: (0, 0)),
        ],
        out_specs=pl.BlockSpec((CB, N, D), lambda b: (b, 0, 0)),
        out_shape=jax.ShapeDtypeStruct((B, N, D), patch.dtype),
        compiler_params=pltpu.CompilerParams(
            dimension_semantics=("parallel",),
        ),
    )(patch, pos_table)


# speedup vs baseline: 2.4456x; 1.0035x over previous
import jax
import jax.numpy as jnp
from jax.experimental import pallas as pl
from jax.experimental.pallas import tpu as pltpu


def _add_kernel(patch_ref, pos_ref, out_ref):
    out_ref[...] = patch_ref[...] + pos_ref[...]


def kernel(patch, pos_table):
    B, N, D = patch.shape
    CB = 8
    return pl.pallas_call(
        _add_kernel,
        grid=(B // CB,),
        in_specs=[
            pl.BlockSpec((CB, N, D), lambda b: (b, 0, 0)),
            pl.BlockSpec((N, D), lambda b: (0, 0)),
        ],
        out_specs=pl.BlockSpec((CB, N, D), lambda b: (b, 0, 0)),
        out_shape=jax.ShapeDtypeStruct((B, N, D), patch.dtype),
        compiler_params=pltpu.CompilerParams(
            dimension_semantics=("arbitrary",),
            vmem_limit_bytes=128 * 1024 * 1024,
        ),
    )(patch, pos_table)
